# baseline (speedup 1.0000x reference)
import functools

import jax
import jax.numpy as jnp
from jax import lax
from jax.experimental import pallas as pl
from jax.experimental.pallas import tpu as pltpu
from jax.experimental.pallas import tpu_sc as plsc

NN = 10000
EE = 320000
DD = 128
HH = 64
NP = 10240

NC = 2
NS = 16
NWK = NC * NS
CH = 128
NCHUNK = 80
EPW = NCHUNK * CH
EPAD = NWK * EPW
RPT = NP // NS

BN = 512


def _make_spmm(width, k_slow, k_fast, slow_cid):
  kmax = max(k_slow, k_fast)
  mesh = plsc.VectorSubcoreMesh(core_axis_name="c", subcore_axis_name="s")

  @functools.partial(
      pl.kernel,
      out_type=jax.ShapeDtypeStruct((NC, NP, width), jnp.float32),
      mesh=mesh,
      compiler_params=pltpu.CompilerParams(use_tc_tiling_on_sc=False),
      scratch_types=[
          pltpu.VMEM((kmax, CH), jnp.int32),
          pltpu.VMEM((kmax, CH), jnp.int32),
          pltpu.VMEM((CH, width), jnp.float32),
          pltpu.VMEM((CH, width), jnp.float32),
          pltpu.VMEM_SHARED((NP, width), jnp.float32),
          pltpu.SemaphoreType.DMA,
          pltpu.SemaphoreType.DMA,
      ],
  )
  def spmm(table, srcs, dsts, zeros, out,
           src_v, dst_v, buf0, buf1, acc, sem0, sem1):
    cid = lax.axis_index("c")
    sid = lax.axis_index("s")
    slow = cid == slow_cid
    k_here = lax.select(slow, k_slow, k_fast)
    base = lax.select(slow, sid * k_slow, NS * k_slow + sid * k_fast)
    pltpu.sync_copy(srcs.at[pl.ds(base, kmax)], src_v)
    pltpu.sync_copy(dsts.at[pl.ds(base, kmax)], dst_v)
    sl = pl.ds(sid * RPT, RPT)
    pltpu.sync_copy(zeros.at[sl], acc.at[sl])
    plsc.subcore_barrier()

    pltpu.async_copy(table.at[src_v.at[0]], buf0, sem0)

    @pl.loop(0, k_here, step=2)
    def _(j):
      pltpu.async_copy(table.at[src_v.at[j + 1]], buf1, sem1)
      pltpu.make_async_copy(table.at[src_v.at[j]], buf0, sem0).wait()
      pltpu.sync_copy(buf0, acc.at[dst_v.at[j]], add=True)

      @pl.when(j + 2 < k_here)
      def _():
        pltpu.async_copy(table.at[src_v.at[j + 2]], buf0, sem0)

      pltpu.make_async_copy(table.at[src_v.at[j + 1]], buf1, sem1).wait()
      pltpu.sync_copy(buf1, acc.at[dst_v.at[j + 1]], add=True)

    plsc.subcore_barrier()
    pltpu.sync_copy(acc.at[sl], out.at[cid, sl])

  return spmm


def _make_deg(k_slow, k_fast, slow_cid):
  kmax = max(k_slow, k_fast)
  mesh = plsc.VectorSubcoreMesh(core_axis_name="c", subcore_axis_name="s")

  @functools.partial(
      pl.kernel,
      out_type=jax.ShapeDtypeStruct((NC, NP, 8), jnp.float32),
      mesh=mesh,
      compiler_params=pltpu.CompilerParams(use_tc_tiling_on_sc=False),
      scratch_types=[
          pltpu.VMEM((kmax, CH), jnp.int32),
          pltpu.VMEM((CH, 8), jnp.float32),
          pltpu.VMEM_SHARED((NP, 8), jnp.float32),
      ],
  )
  def deg(ones, dsts, zeros, out, dst_v, ones_v, acc):
    cid = lax.axis_index("c")
    sid = lax.axis_index("s")
    slow = cid == slow_cid
    k_here = lax.select(slow, k_slow, k_fast)
    base = lax.select(slow, sid * k_slow, NS * k_slow + sid * k_fast)
    pltpu.sync_copy(dsts.at[pl.ds(base, kmax)], dst_v)
    pltpu.sync_copy(ones, ones_v)
    sl = pl.ds(sid * RPT, RPT)
    pltpu.sync_copy(zeros.at[sl], acc.at[sl])
    plsc.subcore_barrier()

    @pl.loop(0, k_here)
    def _(j):
      pltpu.sync_copy(ones_v, acc.at[dst_v.at[j]], add=True)

    plsc.subcore_barrier()
    pltpu.sync_copy(acc.at[sl], out.at[cid, sl])

  return deg


TOTCH = NWK * NCHUNK
FLATCH = TOTCH + 128
SLOW = 1

_spmm64 = _make_spmm(HH, 40, 120, SLOW)
_spmm8 = _make_spmm(8, 70, 90, SLOW)
_deg = _make_deg(74, 86, SLOW)


def _mm1_body(x_ref, w_ref, o_ref):
  o_ref[...] = jnp.dot(x_ref[...], w_ref[...],
                       preferred_element_type=jnp.float32)


def _scale_body(deg_ref, h_ref, ht_ref, dinv_ref):
  deg = deg_ref[0, :, 0] + deg_ref[1, :, 0] + 1.0
  dinv = lax.rsqrt(deg)
  dinv_ref[...] = dinv[:, None]
  ht_ref[...] = h_ref[...] * dinv[:, None]


def _mid_body(a_ref, ht_ref, dinv_ref, b_ref, g_ref, be_ref, w_ref, o_ref):
  s = a_ref[0] + a_ref[1] + ht_ref[...]
  dinv = dinv_ref[...]
  u = s * dinv + b_ref[...]
  mu = jnp.mean(u, axis=1, keepdims=True)
  var = jnp.mean((u - mu) ** 2, axis=1, keepdims=True)
  un = (u - mu) * lax.rsqrt(var + 1e-5) * g_ref[...] + be_ref[...]
  r = jnp.maximum(un, 0.0)
  h = jnp.dot(r, w_ref[...], preferred_element_type=jnp.float32)
  o_ref[...] = h * dinv


def _fin_body(a_ref, ht_ref, dinv_ref, b3_ref, o_ref):
  s = a_ref[0, :, 0] + a_ref[1, :, 0] + ht_ref[:, 0]
  o_ref[...] = (s * dinv_ref[:, 0] + b3_ref[0])[:, None]


def _mm1(xp, W1):
  return pl.pallas_call(
      _mm1_body,
      grid=(NP // BN,),
      in_specs=[
          pl.BlockSpec((BN, DD), lambda i: (i, 0)),
          pl.BlockSpec((DD, HH), lambda i: (0, 0)),
      ],
      out_specs=pl.BlockSpec((BN, HH), lambda i: (i, 0)),
      out_shape=jax.ShapeDtypeStruct((NP, HH), jnp.float32),
  )(xp, W1)


def _scale(deg_out, h1):
  return pl.pallas_call(
      _scale_body,
      grid=(NP // BN,),
      in_specs=[
          pl.BlockSpec((NC, BN, 8), lambda i: (0, i, 0)),
          pl.BlockSpec((BN, HH), lambda i: (i, 0)),
      ],
      out_specs=[
          pl.BlockSpec((BN, HH), lambda i: (i, 0)),
          pl.BlockSpec((BN, 1), lambda i: (i, 0)),
      ],
      out_shape=[
          jax.ShapeDtypeStruct((NP, HH), jnp.float32),
          jax.ShapeDtypeStruct((NP, 1), jnp.float32),
      ],
  )(deg_out, h1)


def _mid(acc, ht, dinv, b, g, be, W, wout):
  return pl.pallas_call(
      _mid_body,
      grid=(NP // BN,),
      in_specs=[
          pl.BlockSpec((NC, BN, HH), lambda i: (0, i, 0)),
          pl.BlockSpec((BN, HH), lambda i: (i, 0)),
          pl.BlockSpec((BN, 1), lambda i: (i, 0)),
          pl.BlockSpec((1, HH), lambda i: (0, 0)),
          pl.BlockSpec((1, HH), lambda i: (0, 0)),
          pl.BlockSpec((1, HH), lambda i: (0, 0)),
          pl.BlockSpec((HH, wout), lambda i: (0, 0)),
      ],
      out_specs=pl.BlockSpec((BN, wout), lambda i: (i, 0)),
      out_shape=jax.ShapeDtypeStruct((NP, wout), jnp.float32),
  )(acc, ht, dinv, b.reshape(1, HH), g.reshape(1, HH), be.reshape(1, HH), W)


def _fin(acc, ht8, dinv, b3):
  return pl.pallas_call(
      _fin_body,
      grid=(NP // BN,),
      in_specs=[
          pl.BlockSpec((NC, BN, 8), lambda i: (0, i, 0)),
          pl.BlockSpec((BN, 8), lambda i: (i, 0)),
          pl.BlockSpec((BN, 1), lambda i: (i, 0)),
          pl.BlockSpec(memory_space=pltpu.SMEM),
      ],
      out_specs=pl.BlockSpec((BN, 1), lambda i: (i, 0)),
      out_shape=jax.ShapeDtypeStruct((NP, 1), jnp.float32),
  )(acc, ht8, dinv, b3)


def kernel(x, edge_index, W1, b1, g1, be1, W2, b2, g2, be2, W3, b3):
  src = edge_index[0]
  dst = edge_index[1]
  srcs = jnp.pad(src, (0, FLATCH * CH - EE)).reshape(FLATCH, CH)
  dsts = jnp.pad(dst, (0, FLATCH * CH - EE),
                 constant_values=NN).reshape(FLATCH, CH)
  xp = jnp.pad(x, ((0, NP - NN), (0, 0)))
  zeros64 = jnp.zeros((NP, HH), jnp.float32)
  zeros8 = jnp.zeros((NP, 8), jnp.float32)
  ones8 = jnp.ones((CH, 8), jnp.float32)
  W3p = jnp.tile(W3, (1, 8))

  deg_out = _deg(ones8, dsts, zeros8)
  h1 = _mm1(xp, W1)
  ht1, dinv = _scale(deg_out, h1)

  acc1 = _spmm64(ht1, srcs, dsts, zeros64)
  ht2 = _mid(acc1, ht1, dinv, b1, g1, be1, W2, HH)

  acc2 = _spmm64(ht2, srcs, dsts, zeros64)
  ht3 = _mid(acc2, ht2, dinv, b2, g2, be2, W3p, 8)

  acc3 = _spmm8(ht3, srcs, dsts, zeros8)
  out = _fin(acc3, ht3, dinv, b3)
  return out[:NN, 0]


# speedup vs baseline: 1.0494x; 1.0494x over previous
import functools

import jax
import jax.numpy as jnp
from jax import lax
from jax.experimental import pallas as pl
from jax.experimental.pallas import tpu as pltpu
from jax.experimental.pallas import tpu_sc as plsc

NN = 10000
EE = 320000
DD = 128
HH = 64
NP = 10240

NC = 2
NS = 16
NWK = NC * NS
CH = 128
NCHUNK = 80
EPW = NCHUNK * CH
EPAD = NWK * EPW
RPT = NP // NS

BN = 512


def _make_spmm(width, k_slow, k_fast, slow_cid):
  kmax = max(k_slow, k_fast)
  mesh = plsc.VectorSubcoreMesh(core_axis_name="c", subcore_axis_name="s")

  @functools.partial(
      pl.kernel,
      out_type=jax.ShapeDtypeStruct((NC, NP, width), jnp.float32),
      mesh=mesh,
      compiler_params=pltpu.CompilerParams(use_tc_tiling_on_sc=False),
      scratch_types=[
          pltpu.VMEM((kmax, CH), jnp.int32),
          pltpu.VMEM((kmax, CH), jnp.int32),
          pltpu.VMEM((CH, width), jnp.float32),
          pltpu.VMEM((CH, width), jnp.float32),
          pltpu.VMEM_SHARED((NP, width), jnp.float32),
          pltpu.SemaphoreType.DMA,
          pltpu.SemaphoreType.DMA,
      ],
  )
  def spmm(table, srcs, dsts, zeros, out,
           src_v, dst_v, buf0, buf1, acc, sem0, sem1):
    cid = lax.axis_index("c")
    sid = lax.axis_index("s")
    slow = cid == slow_cid
    k_here = lax.select(slow, k_slow, k_fast)
    base = lax.select(slow, sid * k_slow, NS * k_slow + sid * k_fast)
    pltpu.sync_copy(srcs.at[pl.ds(base, kmax)], src_v)
    pltpu.sync_copy(dsts.at[pl.ds(base, kmax)], dst_v)
    sl = pl.ds(sid * RPT, RPT)
    pltpu.sync_copy(zeros.at[sl], acc.at[sl])
    plsc.subcore_barrier()

    pltpu.async_copy(table.at[src_v.at[0]], buf0, sem0)

    @pl.loop(0, k_here, step=2)
    def _(j):
      pltpu.async_copy(table.at[src_v.at[j + 1]], buf1, sem1)
      pltpu.make_async_copy(table.at[src_v.at[j]], buf0, sem0).wait()
      pltpu.sync_copy(buf0, acc.at[dst_v.at[j]], add=True)

      @pl.when(j + 2 < k_here)
      def _():
        pltpu.async_copy(table.at[src_v.at[j + 2]], buf0, sem0)

      pltpu.make_async_copy(table.at[src_v.at[j + 1]], buf1, sem1).wait()
      pltpu.sync_copy(buf1, acc.at[dst_v.at[j + 1]], add=True)

    plsc.subcore_barrier()
    pltpu.sync_copy(acc.at[sl], out.at[cid, sl])

  return spmm


def _make_deg(k_slow, k_fast, slow_cid):
  kmax = max(k_slow, k_fast)
  mesh = plsc.VectorSubcoreMesh(core_axis_name="c", subcore_axis_name="s")

  @functools.partial(
      pl.kernel,
      out_type=jax.ShapeDtypeStruct((NC, NP, 8), jnp.float32),
      mesh=mesh,
      compiler_params=pltpu.CompilerParams(use_tc_tiling_on_sc=False),
      scratch_types=[
          pltpu.VMEM((kmax, CH), jnp.int32),
          pltpu.VMEM((CH, 8), jnp.float32),
          pltpu.VMEM_SHARED((NP, 8), jnp.float32),
      ],
  )
  def deg(ones, dsts, zeros, out, dst_v, ones_v, acc):
    cid = lax.axis_index("c")
    sid = lax.axis_index("s")
    slow = cid == slow_cid
    k_here = lax.select(slow, k_slow, k_fast)
    base = lax.select(slow, sid * k_slow, NS * k_slow + sid * k_fast)
    pltpu.sync_copy(dsts.at[pl.ds(base, kmax)], dst_v)
    pltpu.sync_copy(ones, ones_v)
    sl = pl.ds(sid * RPT, RPT)
    pltpu.sync_copy(zeros.at[sl], acc.at[sl])
    plsc.subcore_barrier()

    @pl.loop(0, k_here)
    def _(j):
      pltpu.sync_copy(ones_v, acc.at[dst_v.at[j]], add=True)

    plsc.subcore_barrier()
    pltpu.sync_copy(acc.at[sl], out.at[cid, sl])

  return deg


TOTCH = NWK * NCHUNK
FLATCH = TOTCH + 128
SLOW = 0

_spmm64 = _make_spmm(HH, 40, 120, SLOW)
_spmm8 = _make_spmm(8, 70, 90, SLOW)
_deg = _make_deg(74, 86, SLOW)


def _mm1_body(x_ref, w_ref, o_ref):
  o_ref[...] = jnp.dot(x_ref[...], w_ref[...],
                       preferred_element_type=jnp.float32)


def _scale_body(deg_ref, h_ref, ht_ref, dinv_ref):
  deg = deg_ref[0, :, 0] + deg_ref[1, :, 0] + 1.0
  dinv = lax.rsqrt(deg)
  dinv_ref[...] = dinv[:, None]
  ht_ref[...] = h_ref[...] * dinv[:, None]


def _mid_body(a_ref, ht_ref, dinv_ref, b_ref, g_ref, be_ref, w_ref, o_ref):
  s = a_ref[0] + a_ref[1] + ht_ref[...]
  dinv = dinv_ref[...]
  u = s * dinv + b_ref[...]
  mu = jnp.mean(u, axis=1, keepdims=True)
  var = jnp.mean((u - mu) ** 2, axis=1, keepdims=True)
  un = (u - mu) * lax.rsqrt(var + 1e-5) * g_ref[...] + be_ref[...]
  r = jnp.maximum(un, 0.0)
  h = jnp.dot(r, w_ref[...], preferred_element_type=jnp.float32)
  o_ref[...] = h * dinv


def _fin_body(a_ref, ht_ref, dinv_ref, b3_ref, o_ref):
  s = a_ref[0, :, 0] + a_ref[1, :, 0] + ht_ref[:, 0]
  o_ref[...] = (s * dinv_ref[:, 0] + b3_ref[0])[:, None]


def _mm1(xp, W1):
  return pl.pallas_call(
      _mm1_body,
      grid=(NP // BN,),
      in_specs=[
          pl.BlockSpec((BN, DD), lambda i: (i, 0)),
          pl.BlockSpec((DD, HH), lambda i: (0, 0)),
      ],
      out_specs=pl.BlockSpec((BN, HH), lambda i: (i, 0)),
      out_shape=jax.ShapeDtypeStruct((NP, HH), jnp.float32),
  )(xp, W1)


def _scale(deg_out, h1):
  return pl.pallas_call(
      _scale_body,
      grid=(NP // BN,),
      in_specs=[
          pl.BlockSpec((NC, BN, 8), lambda i: (0, i, 0)),
          pl.BlockSpec((BN, HH), lambda i: (i, 0)),
      ],
      out_specs=[
          pl.BlockSpec((BN, HH), lambda i: (i, 0)),
          pl.BlockSpec((BN, 1), lambda i: (i, 0)),
      ],
      out_shape=[
          jax.ShapeDtypeStruct((NP, HH), jnp.float32),
          jax.ShapeDtypeStruct((NP, 1), jnp.float32),
      ],
  )(deg_out, h1)


def _mid(acc, ht, dinv, b, g, be, W, wout):
  return pl.pallas_call(
      _mid_body,
      grid=(NP // BN,),
      in_specs=[
          pl.BlockSpec((NC, BN, HH), lambda i: (0, i, 0)),
          pl.BlockSpec((BN, HH), lambda i: (i, 0)),
          pl.BlockSpec((BN, 1), lambda i: (i, 0)),
          pl.BlockSpec((1, HH), lambda i: (0, 0)),
          pl.BlockSpec((1, HH), lambda i: (0, 0)),
          pl.BlockSpec((1, HH), lambda i: (0, 0)),
          pl.BlockSpec((HH, wout), lambda i: (0, 0)),
      ],
      out_specs=pl.BlockSpec((BN, wout), lambda i: (i, 0)),
      out_shape=jax.ShapeDtypeStruct((NP, wout), jnp.float32),
  )(acc, ht, dinv, b.reshape(1, HH), g.reshape(1, HH), be.reshape(1, HH), W)


def _fin(acc, ht8, dinv, b3):
  return pl.pallas_call(
      _fin_body,
      grid=(NP // BN,),
      in_specs=[
          pl.BlockSpec((NC, BN, 8), lambda i: (0, i, 0)),
          pl.BlockSpec((BN, 8), lambda i: (i, 0)),
          pl.BlockSpec((BN, 1), lambda i: (i, 0)),
          pl.BlockSpec(memory_space=pltpu.SMEM),
      ],
      out_specs=pl.BlockSpec((BN, 1), lambda i: (i, 0)),
      out_shape=jax.ShapeDtypeStruct((NP, 1), jnp.float32),
  )(acc, ht8, dinv, b3)


def kernel(x, edge_index, W1, b1, g1, be1, W2, b2, g2, be2, W3, b3):
  src = edge_index[0]
  dst = edge_index[1]
  srcs = jnp.pad(src, (0, FLATCH * CH - EE)).reshape(FLATCH, CH)
  dsts = jnp.pad(dst, (0, FLATCH * CH - EE),
                 constant_values=NN).reshape(FLATCH, CH)
  xp = jnp.pad(x, ((0, NP - NN), (0, 0)))
  zeros64 = jnp.zeros((NP, HH), jnp.float32)
  zeros8 = jnp.zeros((NP, 8), jnp.float32)
  ones8 = jnp.ones((CH, 8), jnp.float32)
  W3p = jnp.tile(W3, (1, 8))

  deg_out = _deg(ones8, dsts, zeros8)
  h1 = _mm1(xp, W1)
  ht1, dinv = _scale(deg_out, h1)

  acc1 = _spmm64(ht1, srcs, dsts, zeros64)
  ht2 = _mid(acc1, ht1, dinv, b1, g1, be1, W2, HH)

  acc2 = _spmm64(ht2, srcs, dsts, zeros64)
  ht3 = _mid(acc2, ht2, dinv, b2, g2, be2, W3p, 8)

  acc3 = _spmm8(ht3, srcs, dsts, zeros8)
  out = _fin(acc3, ht3, dinv, b3)
  return out[:NN, 0]


# speedup vs baseline: 2.2075x; 2.1035x over previous
import functools

import jax
import jax.numpy as jnp
from jax import lax
from jax.experimental import pallas as pl
from jax.experimental.pallas import tpu as pltpu
from jax.experimental.pallas import tpu_sc as plsc

NN = 10000
EE = 320000
DD = 128
HH = 64
NP = 10240

NC = 2
NS = 16
NWK = NC * NS
CH = 128
NCHUNK = 80
EPW = NCHUNK * CH
EPAD = NWK * EPW
RPT = NP // NS

BN = 512


def _make_spmm(width, k_slow, k_fast, slow_cid):
  kmax = max(k_slow, k_fast)
  mesh = plsc.VectorSubcoreMesh(core_axis_name="c", subcore_axis_name="s")

  @functools.partial(
      pl.kernel,
      out_type=jax.ShapeDtypeStruct((NC, NP, width), jnp.float32),
      mesh=mesh,
      compiler_params=pltpu.CompilerParams(use_tc_tiling_on_sc=False),
      scratch_types=[
          pltpu.VMEM((kmax, CH), jnp.int32),
          pltpu.VMEM((kmax, CH), jnp.int32),
          pltpu.VMEM((CH, width), jnp.float32),
          pltpu.VMEM((CH, width), jnp.float32),
          pltpu.VMEM_SHARED((NP, width), jnp.float32),
          pltpu.VMEM_SHARED((NP, width), jnp.float32),
          pltpu.SemaphoreType.DMA,
          pltpu.SemaphoreType.DMA,
      ],
  )
  def spmm(table, srcs, dsts, zeros, out,
           src_v, dst_v, buf0, buf1, acc, tab_s, sem0, sem1):
    cid = lax.axis_index("c")
    sid = lax.axis_index("s")
    slow = cid == slow_cid
    k_here = lax.select(slow, k_slow, k_fast)
    base = lax.select(slow, sid * k_slow, NS * k_slow + sid * k_fast)
    pltpu.sync_copy(srcs.at[pl.ds(base, kmax)], src_v)
    pltpu.sync_copy(dsts.at[pl.ds(base, kmax)], dst_v)
    sl = pl.ds(sid * RPT, RPT)
    pltpu.sync_copy(table.at[sl], tab_s.at[sl])
    pltpu.sync_copy(zeros.at[sl], acc.at[sl])
    plsc.subcore_barrier()

    pltpu.async_copy(tab_s.at[src_v.at[0]], buf0, sem0)

    @pl.loop(0, k_here, step=2)
    def _(j):
      pltpu.async_copy(tab_s.at[src_v.at[j + 1]], buf1, sem1)
      pltpu.make_async_copy(tab_s.at[src_v.at[j]], buf0, sem0).wait()
      pltpu.sync_copy(buf0, acc.at[dst_v.at[j]], add=True)

      @pl.when(j + 2 < k_here)
      def _():
        pltpu.async_copy(tab_s.at[src_v.at[j + 2]], buf0, sem0)

      pltpu.make_async_copy(tab_s.at[src_v.at[j + 1]], buf1, sem1).wait()
      pltpu.sync_copy(buf1, acc.at[dst_v.at[j + 1]], add=True)

    plsc.subcore_barrier()
    pltpu.sync_copy(acc.at[sl], out.at[cid, sl])

  return spmm


def _make_deg(k_slow, k_fast, slow_cid):
  kmax = max(k_slow, k_fast)
  mesh = plsc.VectorSubcoreMesh(core_axis_name="c", subcore_axis_name="s")

  @functools.partial(
      pl.kernel,
      out_type=jax.ShapeDtypeStruct((NC, NP, 8), jnp.float32),
      mesh=mesh,
      compiler_params=pltpu.CompilerParams(use_tc_tiling_on_sc=False),
      scratch_types=[
          pltpu.VMEM((kmax, CH), jnp.int32),
          pltpu.VMEM((CH, 8), jnp.float32),
          pltpu.VMEM_SHARED((NP, 8), jnp.float32),
      ],
  )
  def deg(ones, dsts, zeros, out, dst_v, ones_v, acc):
    cid = lax.axis_index("c")
    sid = lax.axis_index("s")
    slow = cid == slow_cid
    k_here = lax.select(slow, k_slow, k_fast)
    base = lax.select(slow, sid * k_slow, NS * k_slow + sid * k_fast)
    pltpu.sync_copy(dsts.at[pl.ds(base, kmax)], dst_v)
    pltpu.sync_copy(ones, ones_v)
    sl = pl.ds(sid * RPT, RPT)
    pltpu.sync_copy(zeros.at[sl], acc.at[sl])
    plsc.subcore_barrier()

    @pl.loop(0, k_here)
    def _(j):
      pltpu.sync_copy(ones_v, acc.at[dst_v.at[j]], add=True)

    plsc.subcore_barrier()
    pltpu.sync_copy(acc.at[sl], out.at[cid, sl])

  return deg


TOTCH = NWK * NCHUNK
FLATCH = TOTCH + 128
SLOW = 0

_spmm64 = _make_spmm(HH, 80, 80, SLOW)
_spmm8 = _make_spmm(8, 80, 80, SLOW)
_deg = _make_deg(80, 80, SLOW)


def _mm1_body(x_ref, w_ref, o_ref):
  o_ref[...] = jnp.dot(x_ref[...], w_ref[...],
                       preferred_element_type=jnp.float32)


def _scale_body(deg_ref, h_ref, ht_ref, dinv_ref):
  deg = deg_ref[0, :, 0] + deg_ref[1, :, 0] + 1.0
  dinv = lax.rsqrt(deg)
  dinv_ref[...] = dinv[:, None]
  ht_ref[...] = h_ref[...] * dinv[:, None]


def _mid_body(a_ref, ht_ref, dinv_ref, b_ref, g_ref, be_ref, w_ref, o_ref):
  s = a_ref[0] + a_ref[1] + ht_ref[...]
  dinv = dinv_ref[...]
  u = s * dinv + b_ref[...]
  mu = jnp.mean(u, axis=1, keepdims=True)
  var = jnp.mean((u - mu) ** 2, axis=1, keepdims=True)
  un = (u - mu) * lax.rsqrt(var + 1e-5) * g_ref[...] + be_ref[...]
  r = jnp.maximum(un, 0.0)
  h = jnp.dot(r, w_ref[...], preferred_element_type=jnp.float32)
  o_ref[...] = h * dinv


def _fin_body(a_ref, ht_ref, dinv_ref, b3_ref, o_ref):
  s = a_ref[0, :, 0] + a_ref[1, :, 0] + ht_ref[:, 0]
  o_ref[...] = (s * dinv_ref[:, 0] + b3_ref[0])[:, None]


def _mm1(xp, W1):
  return pl.pallas_call(
      _mm1_body,
      grid=(NP // BN,),
      in_specs=[
          pl.BlockSpec((BN, DD), lambda i: (i, 0)),
          pl.BlockSpec((DD, HH), lambda i: (0, 0)),
      ],
      out_specs=pl.BlockSpec((BN, HH), lambda i: (i, 0)),
      out_shape=jax.ShapeDtypeStruct((NP, HH), jnp.float32),
  )(xp, W1)


def _scale(deg_out, h1):
  return pl.pallas_call(
      _scale_body,
      grid=(NP // BN,),
      in_specs=[
          pl.BlockSpec((NC, BN, 8), lambda i: (0, i, 0)),
          pl.BlockSpec((BN, HH), lambda i: (i, 0)),
      ],
      out_specs=[
          pl.BlockSpec((BN, HH), lambda i: (i, 0)),
          pl.BlockSpec((BN, 1), lambda i: (i, 0)),
      ],
      out_shape=[
          jax.ShapeDtypeStruct((NP, HH), jnp.float32),
          jax.ShapeDtypeStruct((NP, 1), jnp.float32),
      ],
  )(deg_out, h1)


def _mid(acc, ht, dinv, b, g, be, W, wout):
  return pl.pallas_call(
      _mid_body,
      grid=(NP // BN,),
      in_specs=[
          pl.BlockSpec((NC, BN, HH), lambda i: (0, i, 0)),
          pl.BlockSpec((BN, HH), lambda i: (i, 0)),
          pl.BlockSpec((BN, 1), lambda i: (i, 0)),
          pl.BlockSpec((1, HH), lambda i: (0, 0)),
          pl.BlockSpec((1, HH), lambda i: (0, 0)),
          pl.BlockSpec((1, HH), lambda i: (0, 0)),
          pl.BlockSpec((HH, wout), lambda i: (0, 0)),
      ],
      out_specs=pl.BlockSpec((BN, wout), lambda i: (i, 0)),
      out_shape=jax.ShapeDtypeStruct((NP, wout), jnp.float32),
  )(acc, ht, dinv, b.reshape(1, HH), g.reshape(1, HH), be.reshape(1, HH), W)


def _fin(acc, ht8, dinv, b3):
  return pl.pallas_call(
      _fin_body,
      grid=(NP // BN,),
      in_specs=[
          pl.BlockSpec((NC, BN, 8), lambda i: (0, i, 0)),
          pl.BlockSpec((BN, 8), lambda i: (i, 0)),
          pl.BlockSpec((BN, 1), lambda i: (i, 0)),
          pl.BlockSpec(memory_space=pltpu.SMEM),
      ],
      out_specs=pl.BlockSpec((BN, 1), lambda i: (i, 0)),
      out_shape=jax.ShapeDtypeStruct((NP, 1), jnp.float32),
  )(acc, ht8, dinv, b3)


def kernel(x, edge_index, W1, b1, g1, be1, W2, b2, g2, be2, W3, b3):
  src = edge_index[0]
  dst = edge_index[1]
  srcs = jnp.pad(src, (0, FLATCH * CH - EE)).reshape(FLATCH, CH)
  dsts = jnp.pad(dst, (0, FLATCH * CH - EE),
                 constant_values=NN).reshape(FLATCH, CH)
  xp = jnp.pad(x, ((0, NP - NN), (0, 0)))
  zeros64 = jnp.zeros((NP, HH), jnp.float32)
  zeros8 = jnp.zeros((NP, 8), jnp.float32)
  ones8 = jnp.ones((CH, 8), jnp.float32)
  W3p = jnp.tile(W3, (1, 8))

  deg_out = _deg(ones8, dsts, zeros8)
  h1 = _mm1(xp, W1)
  ht1, dinv = _scale(deg_out, h1)

  acc1 = _spmm64(ht1, srcs, dsts, zeros64)
  ht2 = _mid(acc1, ht1, dinv, b1, g1, be1, W2, HH)

  acc2 = _spmm64(ht2, srcs, dsts, zeros64)
  ht3 = _mid(acc2, ht2, dinv, b2, g2, be2, W3p, 8)

  acc3 = _spmm8(ht3, srcs, dsts, zeros8)
  out = _fin(acc3, ht3, dinv, b3)
  return out[:NN, 0]
